# Initial kernel scaffold; baseline (speedup 1.0000x reference)
import jax
import jax.numpy as jnp
from jax.experimental import pallas as pl


def kernel(x, r_cliques, b_cliques, params):
    raise NotImplementedError("write your pallas kernel here")


# speedup vs baseline: 10.2217x; 10.2217x over previous
import functools

import jax
import jax.numpy as jnp
from jax import lax
from jax.experimental import pallas as pl
from jax.experimental.pallas import tpu as pltpu
from jax.experimental.pallas import tpu_sc as plsc

N_V = 512
HID = 64
NF = 16
NL = 3
CTX = 8
BATCH = 64
N_ENTRIES = N_V * (N_V - 1) // 2

NW = 32
GRAPHS_PER_W = BATCH // NW
NBANDS = 4

_O = [511 * i - i * (i - 1) // 2 for i in range(N_V + 1)]


def _make_bands():
    target = (N_ENTRIES + NBANDS - 1) // NBANDS
    bands = []
    r0 = 0
    for b in range(NBANDS):
        if b == NBANDS - 1:
            r1 = N_V - 1
        else:
            r1 = next(r for r in range(r0 + 1, N_V)
                      if _O[r] - _O[r0] >= target or r == N_V - 1)
        s = (_O[r0] // 8) * 8
        e = ((_O[r1] + 7) // 8) * 8
        bands.append((r0, r1, s, e - s))
        r0 = r1
    return bands


_BANDS = _make_bands()
_BUF = max(nw for _, _, _, nw in _BANDS) + 16


@functools.lru_cache(maxsize=1)
def _make_deg_kernel():
    mesh = plsc.VectorSubcoreMesh(core_axis_name="c", subcore_axis_name="s")

    @functools.partial(
        pl.kernel,
        out_type=jax.ShapeDtypeStruct((BATCH * 16,), jnp.int32),
        mesh=mesh,
        scratch_types=[
            pltpu.VMEM((_BUF,), jnp.int32),
            pltpu.VMEM((N_V + 32,), jnp.int32),
            pltpu.VMEM((N_V * 16,), jnp.int32),
            pltpu.VMEM((GRAPHS_PER_W * 16,), jnp.int32),
        ],
    )
    def deg_kernel(x_hbm, out_hbm, xbuf, colacc, rowacc, st):
        wid = lax.axis_index("s") * 2 + lax.axis_index("c")
        zero16 = jnp.zeros((16,), jnp.int32)
        liota = lax.iota(jnp.int32, 16)

        for g in range(GRAPHS_PER_W):
            gbase = (wid * GRAPHS_PER_W + g) * N_ENTRIES

            for k in range((N_V + 32) // 16):
                colacc[pl.ds(k * 16, 16)] = zero16
            rowacc[pl.ds((N_V - 1) * 16, 16)] = zero16

            for (r0, r1, s, nw) in _BANDS:
                pltpu.sync_copy(x_hbm.at[pl.ds(gbase + s, nw)],
                                xbuf.at[pl.ds(0, nw)])

                def row_body(i, _, s=s):
                    o_i = 511 * i - lax.shift_right_arithmetic(i * (i - 1), 1)
                    p0 = o_i - s
                    length = 511 - i
                    nfull = lax.shift_right_arithmetic(length, 4)
                    rem = length - nfull * 16

                    def t_body(t, acc):
                        xv = xbuf[pl.ds(p0 + t * 16, 16)]
                        base = i + 1 + t * 16
                        colacc[pl.ds(base, 16)] = colacc[pl.ds(base, 16)] + xv
                        return acc + xv

                    acc = lax.fori_loop(0, nfull, t_body, zero16)
                    xv = xbuf[pl.ds(p0 + nfull * 16, 16)]
                    xv = jnp.where(liota < rem, xv, 0)
                    base = i + 1 + nfull * 16
                    colacc[pl.ds(base, 16)] = colacc[pl.ds(base, 16)] + xv
                    acc = acc + xv
                    acc = acc + acc[(liota + 8) & 15]
                    acc = acc + acc[(liota + 4) & 15]
                    acc = acc + acc[(liota + 2) & 15]
                    acc = acc + acc[(liota + 1) & 15]
                    rowacc[pl.ds(i * 16, 16)] = acc
                    return 0

                lax.fori_loop(r0, r1, row_body, 0)

            def blk_body(b, cnt):
                vec = colacc[pl.ds(b * 16, 16)]

                def l_body(l, vec):
                    rv = rowacc[pl.ds((b * 16 + l) * 16, 16)]
                    return vec + jnp.where(liota == l, rv, 0)

                vec = lax.fori_loop(0, 16, l_body, vec)
                return cnt + jnp.where(vec == 0, 1, 0).astype(jnp.int32)

            cnt = lax.fori_loop(0, N_V // 16, blk_body, zero16)
            st[pl.ds(g * 16, 16)] = cnt

        pltpu.sync_copy(st, out_hbm.at[pl.ds(wid * GRAPHS_PER_W * 16,
                                             GRAPHS_PER_W * 16)])

    return deg_kernel


def _tc_body(n0_ref, rcs_ref, bcs_ref,
             w_ne, b_ne, gw0, gb0, gw1, gb1, gw2, gb2, w_ro, b_ro,
             r_wc, r_bc, r_wq, r_bq, r_wk, r_bk, r_wv, r_bv, r_wd, r_bd,
             r_wf, r_bf, r_h1, r_hb1, r_h2t, r_hb2,
             b_wc, b_bc, b_wq, b_bq, b_wk, b_bk, b_wv, b_bv, b_wd, b_bd,
             b_wf, b_bf, b_h1, b_hb1, b_h2t, b_hb2,
             out_ref):
    f32 = jnp.float32

    def mm(a, b):
        return jax.lax.dot_general(a, b, (((1,), (0,)), ((), ())),
                                   preferred_element_type=f32)

    h0 = w_ne[...] + b_ne[...]
    u = h0
    v = h0
    for gw, gb, ind in ((gw0, gb0, NF), (gw1, gb1, HID), (gw2, gb2, HID)):
        wa = gw[pl.ds(0, ind), :]
        wb = gw[pl.ds(ind, ind), :]
        un = jax.nn.relu(mm(u, wa) + mm(u, wb) + gb[...])
        vn = jax.nn.relu(mm(v, wa) + gb[...])
        u, v = un, vn

    n0 = n0_ref[...]
    frac = (jnp.float32(N_V) - n0) * (1.0 / N_V)
    frac0 = n0 * (1.0 / N_V)
    gmean = frac * u + frac0 * v
    gv = mm(gmean, w_ro[...]) + b_ro[...]

    rowid = lax.broadcasted_iota(jnp.int32, (CTX * BATCH, CTX * BATCH), 0)
    colid = lax.broadcasted_iota(jnp.int32, (CTX * BATCH, CTX * BATCH), 1)
    same_g = (rowid % BATCH) == (colid % BATCH)

    def attn(stk, wc, bc, wq, bq, wk, bk, wv, bv, wd, bd):
        e = mm(stk, wc[...]) + bc[...]
        q = mm(e, wq[...]) + bq[...]
        k = mm(e, wk[...]) + bk[...]
        vv = mm(e, wv[...]) + bv[...]
        s = jax.lax.dot_general(q, k, (((1,), (1,)), ((), ())),
                                preferred_element_type=f32)
        s = jnp.where(same_g, s, -1e30)
        m = jnp.max(s, axis=1, keepdims=True)
        p = jnp.exp(s - m)
        a = p / jnp.sum(p, axis=1, keepdims=True)
        o = mm(a, vv)
        emb = bd[...]
        for pos in range(CTX):
            emb = emb + mm(o[pos * BATCH:(pos + 1) * BATCH, :],
                           wd[pl.ds(pos * HID, HID), :])
        return emb

    remb = attn(rcs_ref[...], r_wc, r_bc, r_wq, r_bq, r_wk, r_bk, r_wv, r_bv,
                r_wd, r_bd)
    bemb = attn(bcs_ref[...], b_wc, b_bc, b_wq, b_bq, b_wk, b_bk, b_wv, b_bv,
                b_wd, b_bd)

    def head(emb, wf, bf, h1, hb1, h2t, hb2):
        fr = jax.nn.relu(mm(gv, wf[pl.ds(0, HID), :]) +
                         mm(emb, wf[pl.ds(HID, HID), :]) + bf[...])
        fr2 = jax.nn.relu(mm(fr, h1[...]) + hb1[...])
        return jnp.sum(fr2 * h2t[...], axis=1, keepdims=True) + hb2[...]

    rp = head(remb, r_wf, r_bf, r_h1, r_hb1, r_h2t, r_hb2)
    bp = head(bemb, b_wf, b_bf, b_h1, b_hb1, b_h2t, b_hb2)

    ci = lax.broadcasted_iota(jnp.int32, (BATCH, 2), 1)
    out_ref[...] = jnp.where(ci == 0, rp, bp)


def _branch_weights(params, pre):
    f32 = jnp.float32
    r1 = lambda a: a.astype(f32).reshape(1, -1)
    return (
        params['W_%sc' % pre].astype(f32), r1(params['b_%sc' % pre]),
        params['W_%sq' % pre].astype(f32), r1(params['b_%sq' % pre]),
        params['W_%sk' % pre].astype(f32), r1(params['b_%sk' % pre]),
        params['W_%sv' % pre].astype(f32), r1(params['b_%sv' % pre]),
        params['W_%sd' % pre].astype(f32), r1(params['b_%sd' % pre]),
        params['W_f%s' % pre].astype(f32), r1(params['b_f%s' % pre]),
        params['%sh_W1' % pre].astype(f32), r1(params['%sh_b1' % pre]),
        params['%sh_W2' % pre].astype(f32).reshape(1, HID),
        params['%sh_b2' % pre].astype(f32).reshape(1, 1),
    )


def kernel(x, r_cliques, b_cliques, params):
    f32 = jnp.float32

    x_flat = x.astype(jnp.int32).reshape(BATCH * N_ENTRIES)
    n0_16 = _make_deg_kernel()(x_flat)
    n0 = n0_16.reshape(BATCH, 16).sum(axis=1).astype(f32).reshape(BATCH, 1)

    rcs = r_cliques.astype(f32).transpose(1, 0, 2).reshape(CTX * BATCH, -1)
    bcs = b_cliques.astype(f32).transpose(1, 0, 2).reshape(CTX * BATCH, -1)

    r1 = lambda a: a.astype(f32).reshape(1, -1)
    args = (n0, rcs, bcs,
            params['W_ne'].astype(f32), r1(params['b_ne']),
            params['gnn_W0'].astype(f32), r1(params['gnn_b0']),
            params['gnn_W1'].astype(f32), r1(params['gnn_b1']),
            params['gnn_W2'].astype(f32), r1(params['gnn_b2']),
            params['W_ro'].astype(f32), r1(params['b_ro']),
            *_branch_weights(params, 'r'),
            *_branch_weights(params, 'b'))

    out = pl.pallas_call(
        _tc_body,
        out_shape=jax.ShapeDtypeStruct((BATCH, 2), f32),
    )(*args)
    return out


# speedup vs baseline: 10.3855x; 1.0160x over previous
import functools

import jax
import jax.numpy as jnp
from jax import lax
from jax.experimental import pallas as pl
from jax.experimental.pallas import tpu as pltpu
from jax.experimental.pallas import tpu_sc as plsc

N_V = 512
HID = 64
NF = 16
NL = 3
CTX = 8
BATCH = 64
N_ENTRIES = N_V * (N_V - 1) // 2

NW = 32
GRAPHS_PER_W = BATCH // NW
NBANDS = 4

_O = [511 * i - i * (i - 1) // 2 for i in range(N_V + 1)]


def _make_bands():
    target = (N_ENTRIES + NBANDS - 1) // NBANDS
    bands = []
    r0 = 0
    for b in range(NBANDS):
        if b == NBANDS - 1:
            r1 = N_V - 1
        else:
            r1 = next(r for r in range(r0 + 1, N_V)
                      if _O[r] - _O[r0] >= target or r == N_V - 1)
        s = (_O[r0] // 8) * 8
        e = ((_O[r1] + 7) // 8) * 8
        bands.append((r0, r1, s, e - s))
        r0 = r1
    return bands


_BANDS = _make_bands()
_BUF = max(nw for _, _, _, nw in _BANDS) + 16


@functools.lru_cache(maxsize=1)
def _make_deg_kernel():
    mesh = plsc.VectorSubcoreMesh(core_axis_name="c", subcore_axis_name="s")

    @functools.partial(
        pl.kernel,
        out_type=jax.ShapeDtypeStruct((BATCH * 16,), jnp.int32),
        mesh=mesh,
        scratch_types=[
            pltpu.VMEM((_BUF,), jnp.int32),
            pltpu.VMEM((N_V + 32,), jnp.int32),
            pltpu.VMEM((N_V * 16,), jnp.int32),
            pltpu.VMEM((GRAPHS_PER_W * 16,), jnp.int32),
        ],
    )
    def deg_kernel(x_hbm, out_hbm, xbuf, colacc, rowacc, st):
        wid = lax.axis_index("s") * 2 + lax.axis_index("c")
        zero16 = jnp.zeros((16,), jnp.int32)
        liota = lax.iota(jnp.int32, 16)

        for g in range(GRAPHS_PER_W):
            gbase = (wid * GRAPHS_PER_W + g) * N_ENTRIES

            for k in range((N_V + 32) // 16):
                colacc[pl.ds(k * 16, 16)] = zero16
            rowacc[pl.ds((N_V - 1) * 16, 16)] = zero16

            for (r0, r1, s, nw) in _BANDS:
                pltpu.sync_copy(x_hbm.at[pl.ds(gbase + s, nw)],
                                xbuf.at[pl.ds(0, nw)])

                def row_body(i, _, s=s):
                    o_i = 511 * i - lax.shift_right_arithmetic(i * (i - 1), 1)
                    p0 = o_i - s
                    length = 511 - i
                    nfull = lax.shift_right_arithmetic(length, 4)
                    rem = length - nfull * 16

                    def t_body(t, acc):
                        xv = xbuf[pl.ds(p0 + t * 16, 16)]
                        base = i + 1 + t * 16
                        colacc[pl.ds(base, 16)] = colacc[pl.ds(base, 16)] + xv
                        return acc + xv

                    acc = lax.fori_loop(0, nfull, t_body, zero16)
                    xv = xbuf[pl.ds(p0 + nfull * 16, 16)]
                    xv = jnp.where(liota < rem, xv, 0)
                    base = i + 1 + nfull * 16
                    colacc[pl.ds(base, 16)] = colacc[pl.ds(base, 16)] + xv
                    acc = acc + xv
                    acc = acc + acc[(liota + 8) & 15]
                    acc = acc + acc[(liota + 4) & 15]
                    acc = acc + acc[(liota + 2) & 15]
                    acc = acc + acc[(liota + 1) & 15]
                    rowacc[pl.ds(i * 16, 16)] = acc
                    return 0

                lax.fori_loop(r0, r1, row_body, 0)

            def blk_body(b, cnt):
                vec = colacc[pl.ds(b * 16, 16)]

                def l_body(l, vec):
                    rv = rowacc[pl.ds((b * 16 + l) * 16, 16)]
                    return vec + jnp.where(liota == l, rv, 0)

                vec = lax.fori_loop(0, 16, l_body, vec)
                return cnt + jnp.where(vec == 0, 1, 0).astype(jnp.int32)

            cnt = lax.fori_loop(0, N_V // 16, blk_body, zero16)
            st[pl.ds(g * 16, 16)] = cnt

        pltpu.sync_copy(st, out_hbm.at[pl.ds(wid * GRAPHS_PER_W * 16,
                                             GRAPHS_PER_W * 16)])

    return deg_kernel


def _mm(a, b):
    return jax.lax.dot_general(a, b, (((1,), (0,)), ((), ())),
                               preferred_element_type=jnp.float32)


def _attn_body(rcs_ref, bcs_ref,
               r_wc, r_bc, r_wq, r_bq, r_wk, r_bk, r_wv, r_bv, r_wd, r_bd,
               b_wc, b_bc, b_wq, b_bq, b_wk, b_bk, b_wv, b_bv, b_wd, b_bd,
               out_ref):
    mm = _mm
    rowid = lax.broadcasted_iota(jnp.int32, (CTX * BATCH, CTX * BATCH), 0)
    colid = lax.broadcasted_iota(jnp.int32, (CTX * BATCH, CTX * BATCH), 1)
    same_g = (rowid % BATCH) == (colid % BATCH)

    def attn(stk, wc, bc, wq, bq, wk, bk, wv, bv, wd, bd):
        e = mm(stk, wc[...]) + bc[...]
        q = mm(e, wq[...]) + bq[...]
        k = mm(e, wk[...]) + bk[...]
        vv = mm(e, wv[...]) + bv[...]
        s = jax.lax.dot_general(q, k, (((1,), (1,)), ((), ())),
                                preferred_element_type=jnp.float32)
        s = jnp.where(same_g, s, -1e30)
        m = jnp.max(s, axis=1, keepdims=True)
        p = jnp.exp(s - m)
        a = p / jnp.sum(p, axis=1, keepdims=True)
        o = mm(a, vv)
        emb = bd[...]
        for pos in range(CTX):
            emb = emb + mm(o[pos * BATCH:(pos + 1) * BATCH, :],
                           wd[pl.ds(pos * HID, HID), :])
        return emb

    remb = attn(rcs_ref[...], r_wc, r_bc, r_wq, r_bq, r_wk, r_bk, r_wv, r_bv,
                r_wd, r_bd)
    bemb = attn(bcs_ref[...], b_wc, b_bc, b_wq, b_bq, b_wk, b_bk, b_wv, b_bv,
                b_wd, b_bd)
    out_ref[pl.ds(0, BATCH), :] = remb
    out_ref[pl.ds(BATCH, BATCH), :] = bemb


def _head_body(n0_ref, emb_ref,
               w_ne, b_ne, gw0, gb0, gw1, gb1, gw2, gb2, w_ro, b_ro,
               r_wf, r_bf, r_h1, r_hb1, r_h2t, r_hb2,
               b_wf, b_bf, b_h1, b_hb1, b_h2t, b_hb2,
               out_ref):
    mm = _mm
    h0 = w_ne[...] + b_ne[...]
    u = h0
    v = h0
    for gw, gb, ind in ((gw0, gb0, NF), (gw1, gb1, HID), (gw2, gb2, HID)):
        wa = gw[pl.ds(0, ind), :]
        wb = gw[pl.ds(ind, ind), :]
        un = jax.nn.relu(mm(u, wa) + mm(u, wb) + gb[...])
        vn = jax.nn.relu(mm(v, wa) + gb[...])
        u, v = un, vn

    n0 = n0_ref[...]
    frac = (jnp.float32(N_V) - n0) * (1.0 / N_V)
    frac0 = n0 * (1.0 / N_V)
    gmean = frac * u + frac0 * v
    gv = mm(gmean, w_ro[...]) + b_ro[...]

    def head(emb, wf, bf, h1, hb1, h2t, hb2):
        fr = jax.nn.relu(mm(gv, wf[pl.ds(0, HID), :]) +
                         mm(emb, wf[pl.ds(HID, HID), :]) + bf[...])
        fr2 = jax.nn.relu(mm(fr, h1[...]) + hb1[...])
        return jnp.sum(fr2 * h2t[...], axis=1, keepdims=True) + hb2[...]

    remb = emb_ref[pl.ds(0, BATCH), :]
    bemb = emb_ref[pl.ds(BATCH, BATCH), :]
    rp = head(remb, r_wf, r_bf, r_h1, r_hb1, r_h2t, r_hb2)
    bp = head(bemb, b_wf, b_bf, b_h1, b_hb1, b_h2t, b_hb2)

    ci = lax.broadcasted_iota(jnp.int32, (BATCH, 2), 1)
    out_ref[...] = jnp.where(ci == 0, rp, bp)


def _branch_weights(params, pre):
    f32 = jnp.float32
    r1 = lambda a: a.astype(f32).reshape(1, -1)
    return (
        params['W_%sc' % pre].astype(f32), r1(params['b_%sc' % pre]),
        params['W_%sq' % pre].astype(f32), r1(params['b_%sq' % pre]),
        params['W_%sk' % pre].astype(f32), r1(params['b_%sk' % pre]),
        params['W_%sv' % pre].astype(f32), r1(params['b_%sv' % pre]),
        params['W_%sd' % pre].astype(f32), r1(params['b_%sd' % pre]),
        params['W_f%s' % pre].astype(f32), r1(params['b_f%s' % pre]),
        params['%sh_W1' % pre].astype(f32), r1(params['%sh_b1' % pre]),
        params['%sh_W2' % pre].astype(f32).reshape(1, HID),
        params['%sh_b2' % pre].astype(f32).reshape(1, 1),
    )


def kernel(x, r_cliques, b_cliques, params):
    f32 = jnp.float32

    x_flat = x.astype(jnp.int32).reshape(BATCH * N_ENTRIES)
    n0_16 = _make_deg_kernel()(x_flat)

    rcs = r_cliques.astype(f32).transpose(1, 0, 2).reshape(CTX * BATCH, -1)
    bcs = b_cliques.astype(f32).transpose(1, 0, 2).reshape(CTX * BATCH, -1)

    rb = _branch_weights(params, 'r')
    bb = _branch_weights(params, 'b')

    emb = pl.pallas_call(
        _attn_body,
        out_shape=jax.ShapeDtypeStruct((2 * BATCH, HID), f32),
    )(rcs, bcs, *rb[:10], *bb[:10])

    n0 = n0_16.reshape(BATCH, 16).sum(axis=1).astype(f32).reshape(BATCH, 1)

    r1 = lambda a: a.astype(f32).reshape(1, -1)
    out = pl.pallas_call(
        _head_body,
        out_shape=jax.ShapeDtypeStruct((BATCH, 2), f32),
    )(n0, emb,
      params['W_ne'].astype(f32), r1(params['b_ne']),
      params['gnn_W0'].astype(f32), r1(params['gnn_b0']),
      params['gnn_W1'].astype(f32), r1(params['gnn_b1']),
      params['gnn_W2'].astype(f32), r1(params['gnn_b2']),
      params['W_ro'].astype(f32), r1(params['b_ro']),
      *rb[10:], *bb[10:])
    return out


# speedup vs baseline: 20.1570x; 1.9409x over previous
import functools

import jax
import jax.numpy as jnp
from jax import lax
from jax.experimental import pallas as pl
from jax.experimental.pallas import tpu as pltpu
from jax.experimental.pallas import tpu_sc as plsc

N_V = 512
HID = 64
NF = 16
NL = 3
CTX = 8
BATCH = 64
N_ENTRIES = N_V * (N_V - 1) // 2

NW = 32
GRAPHS_PER_W = BATCH // NW
NBANDS = 4
NBLK = N_V // 16

_O = [511 * i - i * (i - 1) // 2 for i in range(N_V + 1)]


def _make_bands():
    target = (N_ENTRIES + NBANDS - 1) // NBANDS
    bands = []
    i0 = 0
    for b in range(NBANDS):
        if b == NBANDS - 1:
            i1 = NBLK
        else:
            i1 = next(i for i in range(i0 + 1, NBLK + 1)
                      if _O[16 * i] - _O[16 * i0] >= target or i == NBLK)
        s = (_O[16 * i0] // 8) * 8
        e = ((_O[16 * i1] + 7) // 8) * 8
        bands.append((i0, i1, s, e - s))
        i0 = i1
    return bands


_BANDS = _make_bands()
_BUF = max(nw for _, _, _, nw in _BANDS) + 16


@functools.lru_cache(maxsize=1)
def _make_deg_kernel():
    mesh = plsc.VectorSubcoreMesh(core_axis_name="c", subcore_axis_name="s")

    @functools.partial(
        pl.kernel,
        out_type=jax.ShapeDtypeStruct((BATCH * 16,), jnp.int32),
        mesh=mesh,
        scratch_types=[
            pltpu.VMEM((_BUF,), jnp.int32),
            pltpu.VMEM((N_V,), jnp.int32),
            pltpu.VMEM((N_V,), jnp.int32),
            pltpu.VMEM((GRAPHS_PER_W * 16,), jnp.int32),
        ],
    )
    def deg_kernel(x_hbm, out_hbm, xbuf, colacc, rowacc, st):
        wid = lax.axis_index("s") * 2 + lax.axis_index("c")
        zero16 = jnp.zeros((16,), jnp.int32)
        liota = lax.iota(jnp.int32, 16)

        for g in range(GRAPHS_PER_W):
            gbase = (wid * GRAPHS_PER_W + g) * N_ENTRIES

            for k in range(N_V // 16):
                colacc[pl.ds(k * 16, 16)] = zero16

            for (i0, i1, s, nw) in _BANDS:
                pltpu.sync_copy(x_hbm.at[pl.ds(gbase + s, nw)],
                                xbuf.at[pl.ds(0, nw)])

                def blk_body(I, _, s=s):
                    jb = []
                    db = []
                    for l in range(16):
                        i = 16 * I + l
                        o_i = (511 * i -
                               lax.shift_right_arithmetic(i * (i - 1), 1))
                        db.append(o_i - s)
                        jb.append(o_i - i - 1 - s)

                    def jbody(J, racc):
                        b = J * 16
                        cacc = None
                        out = []
                        for l in range(16):
                            ch = xbuf[pl.ds(jb[l] + b, 16)]
                            out.append(racc[l] + ch)
                            cacc = ch if cacc is None else cacc + ch
                        colacc[pl.ds(b, 16)] = colacc[pl.ds(b, 16)] + cacc
                        return tuple(out)

                    racc = list(lax.fori_loop(I + 1, NBLK, jbody,
                                              (zero16,) * 16))

                    dacc = zero16
                    for l in range(15):
                        c0 = xbuf[pl.ds(db[l], 16)]
                        racc[l] = racc[l] + jnp.where(liota <= 14 - l, c0, 0)
                        rot = c0[(liota - (l + 1)) & 15]
                        dacc = dacc + jnp.where(liota >= l + 1, rot, 0)
                    colacc[pl.ds(I * 16, 16)] = (colacc[pl.ds(I * 16, 16)] +
                                                 dacc)

                    rv = zero16
                    for l in range(16):
                        a = racc[l]
                        a = a + a[(liota + 8) & 15]
                        a = a + a[(liota + 4) & 15]
                        a = a + a[(liota + 2) & 15]
                        a = a + a[(liota + 1) & 15]
                        rv = jnp.where(liota == l, a, rv)
                    rowacc[pl.ds(I * 16, 16)] = rv
                    return 0

                lax.fori_loop(i0, i1, blk_body, 0)

            def cnt_body(b, cnt):
                deg = colacc[pl.ds(b * 16, 16)] + rowacc[pl.ds(b * 16, 16)]
                return cnt + jnp.where(deg == 0, 1, 0).astype(jnp.int32)

            cnt = lax.fori_loop(0, NBLK, cnt_body, zero16)
            st[pl.ds(g * 16, 16)] = cnt

        pltpu.sync_copy(st, out_hbm.at[pl.ds(wid * GRAPHS_PER_W * 16,
                                             GRAPHS_PER_W * 16)])

    return deg_kernel


def _mm(a, b):
    return jax.lax.dot_general(a, b, (((1,), (0,)), ((), ())),
                               preferred_element_type=jnp.float32)


def _attn_body(rcs_ref, bcs_ref,
               r_wc, r_bc, r_wq, r_bq, r_wk, r_bk, r_wv, r_bv, r_wd, r_bd,
               b_wc, b_bc, b_wq, b_bq, b_wk, b_bk, b_wv, b_bv, b_wd, b_bd,
               out_ref):
    mm = _mm
    rowid = lax.broadcasted_iota(jnp.int32, (CTX * BATCH, CTX * BATCH), 0)
    colid = lax.broadcasted_iota(jnp.int32, (CTX * BATCH, CTX * BATCH), 1)
    same_g = (rowid % BATCH) == (colid % BATCH)

    def attn(stk, wc, bc, wq, bq, wk, bk, wv, bv, wd, bd):
        e = mm(stk, wc[...]) + bc[...]
        q = mm(e, wq[...]) + bq[...]
        k = mm(e, wk[...]) + bk[...]
        vv = mm(e, wv[...]) + bv[...]
        s = jax.lax.dot_general(q, k, (((1,), (1,)), ((), ())),
                                preferred_element_type=jnp.float32)
        s = jnp.where(same_g, s, -1e30)
        m = jnp.max(s, axis=1, keepdims=True)
        p = jnp.exp(s - m)
        a = p / jnp.sum(p, axis=1, keepdims=True)
        o = mm(a, vv)
        emb = bd[...]
        for pos in range(CTX):
            emb = emb + mm(o[pos * BATCH:(pos + 1) * BATCH, :],
                           wd[pl.ds(pos * HID, HID), :])
        return emb

    remb = attn(rcs_ref[...], r_wc, r_bc, r_wq, r_bq, r_wk, r_bk, r_wv, r_bv,
                r_wd, r_bd)
    bemb = attn(bcs_ref[...], b_wc, b_bc, b_wq, b_bq, b_wk, b_bk, b_wv, b_bv,
                b_wd, b_bd)
    out_ref[pl.ds(0, BATCH), :] = remb
    out_ref[pl.ds(BATCH, BATCH), :] = bemb


def _head_body(n0_ref, emb_ref,
               w_ne, b_ne, gw0, gb0, gw1, gb1, gw2, gb2, w_ro, b_ro,
               r_wf, r_bf, r_h1, r_hb1, r_h2t, r_hb2,
               b_wf, b_bf, b_h1, b_hb1, b_h2t, b_hb2,
               out_ref):
    mm = _mm
    h0 = w_ne[...] + b_ne[...]
    u = h0
    v = h0
    for gw, gb, ind in ((gw0, gb0, NF), (gw1, gb1, HID), (gw2, gb2, HID)):
        wa = gw[pl.ds(0, ind), :]
        wb = gw[pl.ds(ind, ind), :]
        un = jax.nn.relu(mm(u, wa) + mm(u, wb) + gb[...])
        vn = jax.nn.relu(mm(v, wa) + gb[...])
        u, v = un, vn

    n0 = n0_ref[...]
    frac = (jnp.float32(N_V) - n0) * (1.0 / N_V)
    frac0 = n0 * (1.0 / N_V)
    gmean = frac * u + frac0 * v
    gv = mm(gmean, w_ro[...]) + b_ro[...]

    def head(emb, wf, bf, h1, hb1, h2t, hb2):
        fr = jax.nn.relu(mm(gv, wf[pl.ds(0, HID), :]) +
                         mm(emb, wf[pl.ds(HID, HID), :]) + bf[...])
        fr2 = jax.nn.relu(mm(fr, h1[...]) + hb1[...])
        return jnp.sum(fr2 * h2t[...], axis=1, keepdims=True) + hb2[...]

    remb = emb_ref[pl.ds(0, BATCH), :]
    bemb = emb_ref[pl.ds(BATCH, BATCH), :]
    rp = head(remb, r_wf, r_bf, r_h1, r_hb1, r_h2t, r_hb2)
    bp = head(bemb, b_wf, b_bf, b_h1, b_hb1, b_h2t, b_hb2)

    ci = lax.broadcasted_iota(jnp.int32, (BATCH, 2), 1)
    out_ref[...] = jnp.where(ci == 0, rp, bp)


def _branch_weights(params, pre):
    f32 = jnp.float32
    r1 = lambda a: a.astype(f32).reshape(1, -1)
    return (
        params['W_%sc' % pre].astype(f32), r1(params['b_%sc' % pre]),
        params['W_%sq' % pre].astype(f32), r1(params['b_%sq' % pre]),
        params['W_%sk' % pre].astype(f32), r1(params['b_%sk' % pre]),
        params['W_%sv' % pre].astype(f32), r1(params['b_%sv' % pre]),
        params['W_%sd' % pre].astype(f32), r1(params['b_%sd' % pre]),
        params['W_f%s' % pre].astype(f32), r1(params['b_f%s' % pre]),
        params['%sh_W1' % pre].astype(f32), r1(params['%sh_b1' % pre]),
        params['%sh_W2' % pre].astype(f32).reshape(1, HID),
        params['%sh_b2' % pre].astype(f32).reshape(1, 1),
    )


def kernel(x, r_cliques, b_cliques, params):
    f32 = jnp.float32

    x_flat = x.astype(jnp.int32).reshape(BATCH * N_ENTRIES)
    n0_16 = _make_deg_kernel()(x_flat)

    rcs = r_cliques.astype(f32).transpose(1, 0, 2).reshape(CTX * BATCH, -1)
    bcs = b_cliques.astype(f32).transpose(1, 0, 2).reshape(CTX * BATCH, -1)

    rb = _branch_weights(params, 'r')
    bb = _branch_weights(params, 'b')

    emb = pl.pallas_call(
        _attn_body,
        out_shape=jax.ShapeDtypeStruct((2 * BATCH, HID), f32),
    )(rcs, bcs, *rb[:10], *bb[:10])

    n0 = n0_16.reshape(BATCH, 16).sum(axis=1).astype(f32).reshape(BATCH, 1)

    r1 = lambda a: a.astype(f32).reshape(1, -1)
    out = pl.pallas_call(
        _head_body,
        out_shape=jax.ShapeDtypeStruct((BATCH, 2), f32),
    )(n0, emb,
      params['W_ne'].astype(f32), r1(params['b_ne']),
      params['gnn_W0'].astype(f32), r1(params['gnn_b0']),
      params['gnn_W1'].astype(f32), r1(params['gnn_b1']),
      params['gnn_W2'].astype(f32), r1(params['gnn_b2']),
      params['W_ro'].astype(f32), r1(params['b_ro']),
      *rb[10:], *bb[10:])
    return out


# speedup vs baseline: 20.2706x; 1.0056x over previous
import functools

import jax
import jax.numpy as jnp
from jax import lax
from jax.experimental import pallas as pl
from jax.experimental.pallas import tpu as pltpu
from jax.experimental.pallas import tpu_sc as plsc

N_V = 512
HID = 64
NF = 16
NL = 3
CTX = 8
BATCH = 64
N_ENTRIES = N_V * (N_V - 1) // 2

NW = 32
GRAPHS_PER_W = BATCH // NW
NBANDS = 4
NBLK = N_V // 16

_O = [511 * i - i * (i - 1) // 2 for i in range(N_V + 1)]


def _make_bands():
    target = (N_ENTRIES + NBANDS - 1) // NBANDS
    bands = []
    i0 = 0
    for b in range(NBANDS):
        if b == NBANDS - 1:
            i1 = NBLK
        else:
            i1 = next(i for i in range(i0 + 1, NBLK + 1)
                      if _O[16 * i] - _O[16 * i0] >= target or i == NBLK)
        s = (_O[16 * i0] // 8) * 8
        e = ((_O[16 * i1] + 7) // 8) * 8
        bands.append((i0, i1, s, e - s))
        i0 = i1
    return bands


_BANDS = _make_bands()
_BUF = max(nw for _, _, _, nw in _BANDS) + 16


@functools.lru_cache(maxsize=1)
def _make_deg_kernel():
    mesh = plsc.VectorSubcoreMesh(core_axis_name="c", subcore_axis_name="s")

    @functools.partial(
        pl.kernel,
        out_type=jax.ShapeDtypeStruct((BATCH * 16,), jnp.int32),
        mesh=mesh,
        scratch_types=[
            pltpu.VMEM((_BUF,), jnp.int32),
            pltpu.VMEM((N_V,), jnp.int32),
            pltpu.VMEM((N_V,), jnp.int32),
            pltpu.VMEM((GRAPHS_PER_W * 16,), jnp.int32),
        ],
    )
    def deg_kernel(x_hbm, out_hbm, xbuf, colacc, rowacc, st):
        wid = lax.axis_index("s") * 2 + lax.axis_index("c")
        zero16 = jnp.zeros((16,), jnp.int32)
        liota = lax.iota(jnp.int32, 16)

        for g in range(GRAPHS_PER_W):
            gbase = (wid * GRAPHS_PER_W + g) * N_ENTRIES

            for k in range(N_V // 16):
                colacc[pl.ds(k * 16, 16)] = zero16

            for (i0, i1, s, nw) in _BANDS:
                pltpu.sync_copy(x_hbm.at[pl.ds(gbase + s, nw)],
                                xbuf.at[pl.ds(0, nw)])

                def blk_body(I, _, s=s):
                    jb = []
                    db = []
                    for l in range(16):
                        i = 16 * I + l
                        o_i = (511 * i -
                               lax.shift_right_arithmetic(i * (i - 1), 1))
                        db.append(o_i - s)
                        jb.append(o_i - i - 1 - s)

                    def jbody(J, racc):
                        b = J * 16
                        cacc = None
                        out = []
                        for l in range(16):
                            ch = xbuf[pl.ds(jb[l] + b, 16)]
                            out.append(racc[l] + ch)
                            cacc = ch if cacc is None else cacc + ch
                        colacc[pl.ds(b, 16)] = colacc[pl.ds(b, 16)] + cacc
                        return tuple(out)

                    racc = list(lax.fori_loop(I + 1, NBLK, jbody,
                                              (zero16,) * 16))

                    dacc = zero16
                    for l in range(15):
                        c0 = xbuf[pl.ds(db[l], 16)]
                        racc[l] = racc[l] + jnp.where(liota <= 14 - l, c0, 0)
                        rot = c0[(liota - (l + 1)) & 15]
                        dacc = dacc + jnp.where(liota >= l + 1, rot, 0)
                    colacc[pl.ds(I * 16, 16)] = (colacc[pl.ds(I * 16, 16)] +
                                                 dacc)

                    rv = zero16
                    for l in range(16):
                        a = racc[l]
                        a = a + a[(liota + 8) & 15]
                        a = a + a[(liota + 4) & 15]
                        a = a + a[(liota + 2) & 15]
                        a = a + a[(liota + 1) & 15]
                        rv = jnp.where(liota == l, a, rv)
                    rowacc[pl.ds(I * 16, 16)] = rv
                    return 0

                lax.fori_loop(i0, i1, blk_body, 0)

            def cnt_body(b, cnt):
                deg = colacc[pl.ds(b * 16, 16)] + rowacc[pl.ds(b * 16, 16)]
                return cnt + jnp.where(deg == 0, 1, 0).astype(jnp.int32)

            cnt = lax.fori_loop(0, NBLK, cnt_body, zero16)
            st[pl.ds(g * 16, 16)] = cnt

        pltpu.sync_copy(st, out_hbm.at[pl.ds(wid * GRAPHS_PER_W * 16,
                                             GRAPHS_PER_W * 16)])

    return deg_kernel


def _mm(a, b):
    return jax.lax.dot_general(a, b, (((1,), (0,)), ((), ())),
                               preferred_element_type=jnp.float32)


def _attn_body(rcs_ref, bcs_ref,
               r_wc, r_bc, r_wq, r_bq, r_wk, r_bk, r_wv, r_bv, r_wd, r_bd,
               b_wc, b_bc, b_wq, b_bq, b_wk, b_bk, b_wv, b_bv, b_wd, b_bd,
               out_ref):
    mm = _mm
    rowid = lax.broadcasted_iota(jnp.int32, (CTX * BATCH, CTX * BATCH), 0)
    colid = lax.broadcasted_iota(jnp.int32, (CTX * BATCH, CTX * BATCH), 1)
    same_g = (rowid // CTX) == (colid // CTX)

    def attn(stk, wc, bc, wq, bq, wk, bk, wv, bv, wd, bd):
        e = mm(stk, wc[...]) + bc[...]
        q = mm(e, wq[...]) + bq[...]
        k = mm(e, wk[...]) + bk[...]
        vv = mm(e, wv[...]) + bv[...]
        s = jax.lax.dot_general(q, k, (((1,), (1,)), ((), ())),
                                preferred_element_type=jnp.float32)
        s = jnp.where(same_g, s, -1e30)
        m = jnp.max(s, axis=1, keepdims=True)
        p = jnp.exp(s - m)
        a = p / jnp.sum(p, axis=1, keepdims=True)
        o = mm(a, vv)
        gid = lax.broadcasted_iota(jnp.int32, (BATCH, CTX * BATCH), 0)
        rid = lax.broadcasted_iota(jnp.int32, (BATCH, CTX * BATCH), 1)
        emb = bd[...]
        for pos in range(CTX):
            sel = (rid == gid * CTX + pos).astype(jnp.float32)
            emb = emb + mm(sel, mm(o, wd[pl.ds(pos * HID, HID), :]))
        return emb

    remb = attn(rcs_ref[...], r_wc, r_bc, r_wq, r_bq, r_wk, r_bk, r_wv, r_bv,
                r_wd, r_bd)
    bemb = attn(bcs_ref[...], b_wc, b_bc, b_wq, b_bq, b_wk, b_bk, b_wv, b_bv,
                b_wd, b_bd)
    out_ref[pl.ds(0, BATCH), :] = remb
    out_ref[pl.ds(BATCH, BATCH), :] = bemb


def _head_body(n0_ref, emb_ref,
               w_ne, b_ne, gw0, gb0, gw1, gb1, gw2, gb2, w_ro, b_ro,
               r_wf, r_bf, r_h1, r_hb1, r_h2t, r_hb2,
               b_wf, b_bf, b_h1, b_hb1, b_h2t, b_hb2,
               out_ref):
    mm = _mm
    h0 = w_ne[...] + b_ne[...]
    u = h0
    v = h0
    for gw, gb, ind in ((gw0, gb0, NF), (gw1, gb1, HID), (gw2, gb2, HID)):
        wa = gw[pl.ds(0, ind), :]
        wb = gw[pl.ds(ind, ind), :]
        un = jax.nn.relu(mm(u, wa) + mm(u, wb) + gb[...])
        vn = jax.nn.relu(mm(v, wa) + gb[...])
        u, v = un, vn

    n0 = jnp.sum(n0_ref[...].astype(jnp.float32), axis=1,
                 keepdims=True)
    frac = (jnp.float32(N_V) - n0) * (1.0 / N_V)
    frac0 = n0 * (1.0 / N_V)
    gmean = frac * u + frac0 * v
    gv = mm(gmean, w_ro[...]) + b_ro[...]

    def head(emb, wf, bf, h1, hb1, h2t, hb2):
        fr = jax.nn.relu(mm(gv, wf[pl.ds(0, HID), :]) +
                         mm(emb, wf[pl.ds(HID, HID), :]) + bf[...])
        fr2 = jax.nn.relu(mm(fr, h1[...]) + hb1[...])
        return jnp.sum(fr2 * h2t[...], axis=1, keepdims=True) + hb2[...]

    remb = emb_ref[pl.ds(0, BATCH), :]
    bemb = emb_ref[pl.ds(BATCH, BATCH), :]
    rp = head(remb, r_wf, r_bf, r_h1, r_hb1, r_h2t, r_hb2)
    bp = head(bemb, b_wf, b_bf, b_h1, b_hb1, b_h2t, b_hb2)

    ci = lax.broadcasted_iota(jnp.int32, (BATCH, 2), 1)
    out_ref[...] = jnp.where(ci == 0, rp, bp)


def _branch_weights(params, pre):
    f32 = jnp.float32
    r1 = lambda a: a.astype(f32).reshape(1, -1)
    return (
        params['W_%sc' % pre].astype(f32), r1(params['b_%sc' % pre]),
        params['W_%sq' % pre].astype(f32), r1(params['b_%sq' % pre]),
        params['W_%sk' % pre].astype(f32), r1(params['b_%sk' % pre]),
        params['W_%sv' % pre].astype(f32), r1(params['b_%sv' % pre]),
        params['W_%sd' % pre].astype(f32), r1(params['b_%sd' % pre]),
        params['W_f%s' % pre].astype(f32), r1(params['b_f%s' % pre]),
        params['%sh_W1' % pre].astype(f32), r1(params['%sh_b1' % pre]),
        params['%sh_W2' % pre].astype(f32).reshape(1, HID),
        params['%sh_b2' % pre].astype(f32).reshape(1, 1),
    )


def kernel(x, r_cliques, b_cliques, params):
    f32 = jnp.float32

    x_flat = x.astype(jnp.int32).reshape(BATCH * N_ENTRIES)
    n0_16 = _make_deg_kernel()(x_flat)

    rcs = r_cliques.astype(f32).reshape(BATCH * CTX, -1)
    bcs = b_cliques.astype(f32).reshape(BATCH * CTX, -1)

    rb = _branch_weights(params, 'r')
    bb = _branch_weights(params, 'b')

    emb = pl.pallas_call(
        _attn_body,
        out_shape=jax.ShapeDtypeStruct((2 * BATCH, HID), f32),
    )(rcs, bcs, *rb[:10], *bb[:10])

    r1 = lambda a: a.astype(f32).reshape(1, -1)
    out = pl.pallas_call(
        _head_body,
        out_shape=jax.ShapeDtypeStruct((BATCH, 2), f32),
    )(n0_16.reshape(BATCH, 16), emb,
      params['W_ne'].astype(f32), r1(params['b_ne']),
      params['gnn_W0'].astype(f32), r1(params['gnn_b0']),
      params['gnn_W1'].astype(f32), r1(params['gnn_b1']),
      params['gnn_W2'].astype(f32), r1(params['gnn_b2']),
      params['W_ro'].astype(f32), r1(params['b_ro']),
      *rb[10:], *bb[10:])
    return out
